# Initial kernel scaffold; baseline (speedup 1.0000x reference)
import jax
import jax.numpy as jnp
from jax.experimental import pallas as pl


def kernel(x, W):
    raise NotImplementedError("write your pallas kernel here")


# speedup vs baseline: 1.5007x; 1.5007x over previous
import jax
import jax.numpy as jnp
from jax.experimental import pallas as pl
from jax.experimental.pallas import tpu as pltpu

_HIDDEN = 4096
_EXPERTS = 64
_K = 8
_BLOCK = 512


def _router_block(x_ref, w_ref, tw_ref, te_ref):
    x = x_ref[...]
    w = w_ref[...]
    logits = jax.lax.dot_general(
        w, x, (((1,), (1,)), ((), ())), preferred_element_type=jnp.float32)
    eidx = jax.lax.broadcasted_iota(jnp.int32, logits.shape, 0)
    l = logits
    vals = []
    idxs = []
    for _ in range(_K):
        m = jnp.max(l, axis=0, keepdims=True)
        idx = jnp.min(jnp.where(l == m, eidx, _EXPERTS),
                      axis=0, keepdims=True)
        vals.append(m)
        idxs.append(idx)
        l = jnp.where(eidx == idx, -jnp.inf, l)
    v = jnp.concatenate(vals, axis=0)
    e = jnp.exp(v - v[0:1])
    wts = e / jnp.sum(e, axis=0, keepdims=True)
    tw_ref[...] = wts.T
    te_ref[...] = jnp.concatenate(idxs, axis=0).T


def kernel(x, W):
    tokens = x.shape[0]
    grid = (tokens // _BLOCK,)
    tw, te = pl.pallas_call(
        _router_block,
        grid=grid,
        in_specs=[
            pl.BlockSpec((_BLOCK, _HIDDEN), lambda i: (i, 0)),
            pl.BlockSpec((_EXPERTS, _HIDDEN), lambda i: (0, 0)),
        ],
        out_specs=[
            pl.BlockSpec((_BLOCK, _K), lambda i: (i, 0)),
            pl.BlockSpec((_BLOCK, _K), lambda i: (i, 0)),
        ],
        out_shape=[
            jax.ShapeDtypeStruct((tokens, _K), jnp.float32),
            jax.ShapeDtypeStruct((tokens, _K), jnp.int32),
        ],
    )(x, W)
    return tw, te


# speedup vs baseline: 1.6162x; 1.0769x over previous
import jax
import jax.numpy as jnp
from jax.experimental import pallas as pl
from jax.experimental.pallas import tpu as pltpu

_HIDDEN = 4096
_EXPERTS = 64
_K = 8
_BLOCK = 1024


def _router_block(x_ref, w_ref, tw_ref, te_ref):
    x = x_ref[...]
    w = w_ref[...]
    logits = jax.lax.dot_general(
        w, x, (((1,), (1,)), ((), ())), preferred_element_type=jnp.float32)
    eidx = jax.lax.broadcasted_iota(jnp.int32, logits.shape, 0)
    l = logits
    vals = []
    idxs = []
    for _ in range(_K):
        m = jnp.max(l, axis=0, keepdims=True)
        idx = jnp.min(jnp.where(l == m, eidx, _EXPERTS),
                      axis=0, keepdims=True)
        vals.append(m)
        idxs.append(idx)
        l = jnp.where(eidx == idx, -jnp.inf, l)
    v = jnp.concatenate(vals, axis=0)
    e = jnp.exp(v - v[0:1])
    wts = e / jnp.sum(e, axis=0, keepdims=True)
    tw_ref[...] = wts.T
    te_ref[...] = jnp.concatenate(idxs, axis=0).T


def kernel(x, W):
    tokens = x.shape[0]
    grid = (tokens // _BLOCK,)
    tw, te = pl.pallas_call(
        _router_block,
        grid=grid,
        in_specs=[
            pl.BlockSpec((_BLOCK, _HIDDEN), lambda i: (i, 0)),
            pl.BlockSpec((_EXPERTS, _HIDDEN), lambda i: (0, 0)),
        ],
        out_specs=[
            pl.BlockSpec((_BLOCK, _K), lambda i: (i, 0)),
            pl.BlockSpec((_BLOCK, _K), lambda i: (i, 0)),
        ],
        out_shape=[
            jax.ShapeDtypeStruct((tokens, _K), jnp.float32),
            jax.ShapeDtypeStruct((tokens, _K), jnp.int32),
        ],
    )(x, W)
    return tw, te
